# baseline (speedup 1.0000x reference)
import functools
import math

import jax
import jax.numpy as jnp
from jax import lax
from jax.experimental import pallas as pl
from jax.experimental.pallas import tpu as pltpu
from jax.experimental.pallas import tpu_sc as plsc

D = 1024
Z = 256
C = 128
B = 2
L = 4096
NCH = L // C
BL = B * L
NEG = -50000.0

N_DIM_CONST = 2

NC = 2
NS = 16
NW = NC * NS
GCH = 64


def _tri():
    r = lax.broadcasted_iota(jnp.int32, (C, C), 0)
    c = lax.broadcasted_iota(jnp.int32, (C, C), 1)
    return (c <= r).astype(jnp.float32)


def _k1_body(x_ref, a_ref, p_ref, pq_ref, cb_ref, om_ref, wcs_ref,
             mx_ref, g2_ref, sidx_ref, carry_ref, cnt_ref):
    j = pl.program_id(1)

    @pl.when(j == 0)
    def _():
        carry_ref[...] = jnp.zeros_like(carry_ref)
        cnt_ref[0] = 0.0

    x = x_ref[0]
    x2 = jnp.concatenate([x, x], axis=1)
    tri = _tri()
    y = x2 * a_ref[...]
    cum = jnp.dot(tri, y, preferred_element_type=jnp.float32)
    s = cum * p_ref[...] + carry_ref[0:1, :] * pq_ref[...]
    carry_ref[0:1, :] = s[C - 1:C, :]
    sc = s * cb_ref[...]
    conv = sc[:, :D] + sc[:, D:]
    mx = conv + x * om_ref[...]
    mx = mx * jax.nn.sigmoid(mx)
    mx_ref[0, 0] = mx

    logit = jnp.sum(mx * wcs_ref[...], axis=1, keepdims=True)
    sel = (logit > 0.0).astype(jnp.float32)
    g2_ref[0] = jax.nn.sigmoid(logit) * sel
    csel = jnp.dot(tri, sel, preferred_element_type=jnp.float32)
    sidx_ref[0] = (csel + cnt_ref[0]) * sel
    cnt_ref[0] = cnt_ref[0] + csel[C - 1, 0]


def _run_k1(x, a, p, pq, cb, om, wcs):
    full = lambda s: pl.BlockSpec(s, lambda b, j: (0,) * len(s))
    return pl.pallas_call(
        _k1_body,
        grid=(B, NCH),
        in_specs=[
            pl.BlockSpec((1, C, D), lambda b, j: (b, j, 0)),
            full((C, 2 * D)), full((C, 2 * D)), full((C, 2 * D)),
            full((1, 2 * D)), full((1, D)), full((1, D)),
        ],
        out_specs=[
            pl.BlockSpec((1, 1, C, D), lambda b, j: (b, j, 0, 0)),
            pl.BlockSpec((1, C, 1), lambda b, j: (b, j, 0)),
            pl.BlockSpec((1, C, 1), lambda b, j: (b, j, 0)),
        ],
        out_shape=[
            jax.ShapeDtypeStruct((B, NCH, C, D), jnp.float32),
            jax.ShapeDtypeStruct((B, L, 1), jnp.float32),
            jax.ShapeDtypeStruct((B, L, 1), jnp.float32),
        ],
        scratch_shapes=[
            pltpu.VMEM((8, 2 * D), jnp.float32),
            pltpu.SMEM((1,), jnp.float32),
        ],
        compiler_params=pltpu.CompilerParams(
            dimension_semantics=("arbitrary", "arbitrary")),
    )(x, a, p, pq, cb, om, wcs)


def _k2_body(x_ref, mx_ref, wq_ref, wk_ref, wv_ref, zq_ref, zk_ref, v_ref):
    m = mx_ref[...].astype(jnp.bfloat16)
    zq_ref[...] = jnp.dot(m, wq_ref[...].astype(jnp.bfloat16),
                          preferred_element_type=jnp.float32
                          ).astype(jnp.bfloat16)
    zk_ref[...] = jnp.dot(m, wk_ref[...].astype(jnp.bfloat16),
                          preferred_element_type=jnp.float32
                          ).astype(jnp.bfloat16)
    vv = jnp.dot(x_ref[...].astype(jnp.bfloat16),
                 wv_ref[...].astype(jnp.bfloat16),
                 preferred_element_type=jnp.float32)
    v_ref[...] = (vv * jax.nn.sigmoid(vv)).astype(jnp.bfloat16)


def _run_k2(x2, mx2, wq, wk, wv):
    R = 512
    full = lambda s: pl.BlockSpec(s, lambda i: (0,) * len(s))
    return pl.pallas_call(
        _k2_body,
        grid=(BL // R,),
        in_specs=[
            pl.BlockSpec((R, D), lambda i: (i, 0)),
            pl.BlockSpec((R, D), lambda i: (i, 0)),
            full((D, Z)), full((D, Z)), full((D, D)),
        ],
        out_specs=[
            pl.BlockSpec((R, Z), lambda i: (i, 0)),
            pl.BlockSpec((R, Z), lambda i: (i, 0)),
            pl.BlockSpec((R, D), lambda i: (i, 0)),
        ],
        out_shape=[
            jax.ShapeDtypeStruct((BL, Z), jnp.bfloat16),
            jax.ShapeDtypeStruct((BL, Z), jnp.bfloat16),
            jax.ShapeDtypeStruct((BL, D), jnp.bfloat16),
        ],
        compiler_params=pltpu.CompilerParams(
            dimension_semantics=("arbitrary",)),
    )(x2, mx2, wq, wk, wv)


def _sc1_body(sidx_hbm, pos_hbm, idx_v, pos_v):
    wid = lax.axis_index("c") * NS + lax.axis_index("s")
    for b in range(B):
        @pl.when(wid == b)
        def _():
            pltpu.sync_copy(sidx_hbm.at[b], idx_v)

            def zf(i, c):
                pos_v[pl.ds(i * 16, 16)] = jnp.zeros((16,), jnp.int32)
                return c
            lax.fori_loop(0, L // 16, zf, 0)

            base = b * L

            def scat(i, c):
                ranks = idx_v[pl.ds(i * 16, 16)]
                tvec = lax.iota(jnp.int32, 16) + (i * 16 + base)
                plsc.store_scatter(pos_v, [jnp.maximum(ranks - 1, 0)],
                                   tvec, mask=ranks > 0)
                return c
            lax.fori_loop(0, L // 16, scat, 0)
            pltpu.sync_copy(pos_v, pos_hbm.at[b])


def _run_sc1(sidx_i):
    mesh = plsc.VectorSubcoreMesh(core_axis_name="c", subcore_axis_name="s")
    return pl.kernel(
        _sc1_body,
        out_type=jax.ShapeDtypeStruct((B, L), jnp.int32),
        mesh=mesh,
        scratch_types=[
            pltpu.VMEM((L,), jnp.int32),
            pltpu.VMEM((L,), jnp.int32),
        ],
        compiler_params=pltpu.CompilerParams(needs_layout_passes=False,
                                             use_tc_tiling_on_sc=False),
    )(sidx_i)


def _scgather_body(meta, idx_hbm, *rest):
    nsrc = len(meta)
    srcs = rest[:nsrc]
    outs = rest[nsrc:2 * nsrc]
    idx_v = rest[2 * nsrc]
    bufs = rest[2 * nsrc + 1:2 * nsrc + 1 + 2 * nsrc]
    sem = rest[-1]
    cid = lax.axis_index("c")
    sid = lax.axis_index("s")
    for t, (w, lo, hi) in enumerate(meta):
        ntiles = (hi - lo) * NC
        rpt = BL // ntiles
        nch = rpt // GCH
        src, out, b0, b1 = srcs[t], outs[t], bufs[2 * t], bufs[2 * t + 1]

        @pl.when((sid >= lo) & (sid < hi))
        def _(src=src, out=out, b0=b0, b1=b1, lo=lo, rpt=rpt, nch=nch):
            r = (sid - lo) * NC + cid
            row0 = r * rpt
            pltpu.sync_copy(idx_hbm.at[pl.ds(row0 // GCH, nch)],
                            idx_v.at[pl.ds(0, nch)])
            pltpu.async_copy(src.at[idx_v.at[0]], b0, sem)

            def step(i, carry):
                for par, (cur, nxt) in enumerate(((b0, b1), (b1, b0))):
                    @pl.when(lax.rem(i, 2) == par)
                    def _(cur=cur, nxt=nxt):
                        @pl.when(i + 1 < nch)
                        def _():
                            pltpu.async_copy(src.at[idx_v.at[i + 1]], nxt,
                                             sem)
                        pltpu.make_async_copy(src.at[idx_v.at[i]], cur,
                                              sem).wait()
                        pltpu.sync_copy(cur,
                                        out.at[pl.ds(row0 + i * GCH, GCH)])
                return carry
            lax.fori_loop(0, nch, step, 0)


def _run_scgather(idx2d, srcs, meta):
    mesh = plsc.VectorSubcoreMesh(core_axis_name="c", subcore_axis_name="s")
    return pl.kernel(
        functools.partial(_scgather_body, meta),
        out_type=tuple(jax.ShapeDtypeStruct((BL, w), jnp.bfloat16)
                       for (w, lo, hi) in meta),
        mesh=mesh,
        scratch_types=(
            [pltpu.VMEM((32, GCH), jnp.int32)]
            + [pltpu.VMEM((GCH, w), jnp.bfloat16)
               for (w, lo, hi) in meta for _ in range(2)]
            + [pltpu.SemaphoreType.DMA]
        ),
        compiler_params=pltpu.CompilerParams(needs_layout_passes=False,
                                             use_tc_tiling_on_sc=False),
    )(idx2d, *srcs)


def _k3_body(czq_ref, czk_ref, cv_ref, bias_ref, wh_ref, counts_ref, chw_ref):
    b = pl.program_id(0)
    j = pl.program_id(1)
    q = czq_ref[0, 0]
    k = czk_ref[0, 0]
    s = lax.dot_general(q, k, (((1,), (1,)), ((), ())),
                        preferred_element_type=jnp.float32)
    s = s * (1.0 / math.sqrt(Z)) + bias_ref[...]
    col = lax.broadcasted_iota(jnp.int32, (C, C), 1) + j * C
    s = jnp.where(col < counts_ref[b], s, NEG)
    m = jnp.max(s, axis=-1, keepdims=True)
    e = jnp.exp(s - m)
    attn = (e / jnp.sum(e, axis=-1, keepdims=True)).astype(jnp.bfloat16)
    h = jnp.dot(attn, cv_ref[0, 0], preferred_element_type=jnp.float32)
    chw_ref[0, 0] = jnp.dot(h.astype(jnp.bfloat16),
                            wh_ref[...].astype(jnp.bfloat16),
                            preferred_element_type=jnp.float32
                            ).astype(jnp.bfloat16)


def _run_k3(czq, czk, cv, bias, wh, counts):
    full = lambda s: pl.BlockSpec(s, lambda b, j: (0,) * len(s))
    return pl.pallas_call(
        _k3_body,
        grid=(B, NCH),
        in_specs=[
            pl.BlockSpec((1, 1, C, Z), lambda b, j: (b, j, 0, 0)),
            pl.BlockSpec((1, 1, C, Z), lambda b, j: (b, j, 0, 0)),
            pl.BlockSpec((1, 1, C, D), lambda b, j: (b, j, 0, 0)),
            full((C, C)), full((D, D)),
            pl.BlockSpec(memory_space=pltpu.SMEM),
        ],
        out_specs=pl.BlockSpec((1, 1, C, D), lambda b, j: (b, j, 0, 0)),
        out_shape=jax.ShapeDtypeStruct((B, NCH, C, D), jnp.bfloat16),
        compiler_params=pltpu.CompilerParams(
            dimension_semantics=("arbitrary", "arbitrary")),
    )(czq, czk, cv, bias, wh, counts)


def _k4_body(x_ref, hxw_ref, g2_ref, out_ref):
    g = g2_ref[...]
    hw = hxw_ref[...].astype(jnp.float32)
    out_ref[...] = jnp.where(g > 0.0, x_ref[...] + hw * g, x_ref[...])


def _run_k4(x2, hxw, g2):
    R = 512
    return pl.pallas_call(
        _k4_body,
        grid=(BL // R,),
        in_specs=[
            pl.BlockSpec((R, D), lambda i: (i, 0)),
            pl.BlockSpec((R, D), lambda i: (i, 0)),
            pl.BlockSpec((R, 1), lambda i: (i, 0)),
        ],
        out_specs=pl.BlockSpec((R, D), lambda i: (i, 0)),
        out_shape=jax.ShapeDtypeStruct((BL, D), jnp.float32),
        compiler_params=pltpu.CompilerParams(
            dimension_semantics=("arbitrary",)),
    )(x2, hxw, g2)


def kernel(x, delta, alpha, beta, gamma, omega, temp, w_conf, w_q, w_k,
           w_v, w_h, rel_pos_bias):
    p = jax.nn.sigmoid(delta[:, :, 0])
    aa = jax.nn.sigmoid(alpha[:, :, 0])
    qd = 1.0 - p * aa
    logq = jnp.log(qd)
    cc = p * beta[:, :, 0] * gamma * math.sqrt(1.0 / N_DIM_CONST)
    t = jnp.arange(C, dtype=jnp.float32)[:, None, None]
    apow = jnp.exp(-t * logq[None])
    ppow = jnp.exp(t * logq[None])
    pq = ppow * qd[None]
    to2d = lambda z: jnp.concatenate([z[:, :, 0], z[:, :, 1]], axis=1)
    a2 = to2d(apow)
    p2 = to2d(ppow)
    pq2 = to2d(pq)
    cb = jnp.concatenate([cc[:, 0], cc[:, 1]])[None, :]
    om = omega[None, :]
    wcs = (w_conf[:, 0] / jnp.exp(temp[0]))[None, :]

    mx4, g2, sidx = _run_k1(x, a2, p2, pq2, cb, om, wcs)
    mx2 = mx4.reshape(BL, D)
    x2 = x.reshape(BL, D)

    zq2, zk2, v2 = _run_k2(x2, mx2, w_q, w_k, w_v)

    sidx_i = sidx[:, :, 0].astype(jnp.int32)
    counts = jnp.max(sidx_i, axis=1)

    pos_g = _run_sc1(sidx_i)
    pos2d = pos_g.reshape(BL // GCH, GCH)

    czq2, czk2, cv2 = _run_scgather(
        pos2d, (zq2, zk2, v2),
        ((Z, 8, 12), (Z, 12, 16), (D, 0, 8)))

    off = jnp.arange(C)[:, None] - jnp.arange(C)[None, :] + C - 1
    bias = rel_pos_bias[off]
    chw = _run_k3(czq2.reshape(B, NCH, C, Z), czk2.reshape(B, NCH, C, Z),
                  cv2.reshape(B, NCH, C, D), bias, w_h, counts)
    chw2 = chw.reshape(BL, D)

    boff = (jnp.arange(B, dtype=jnp.int32) * L)[:, None]
    eidx = boff + jnp.maximum(sidx_i - 1, 0)
    (hxw,) = _run_scgather(eidx.reshape(BL // GCH, GCH), (chw2,),
                           ((D, 0, 16),))

    out2 = _run_k4(x2, hxw, g2.reshape(BL, 1))
    return out2.reshape(B, L, D)


# speedup vs baseline: 1.1158x; 1.1158x over previous
import functools
import math

import jax
import jax.numpy as jnp
from jax import lax
from jax.experimental import pallas as pl
from jax.experimental.pallas import tpu as pltpu
from jax.experimental.pallas import tpu_sc as plsc

D = 1024
Z = 256
C = 128
B = 2
L = 4096
NCH = L // C
BL = B * L
NEG = -50000.0

N_DIM_CONST = 2
_STAGE = 5

NC = 2
NS = 16
NW = NC * NS
GCH = 64
GCH_LOG2 = 6


def _tri():
    r = lax.broadcasted_iota(jnp.int32, (C, C), 0)
    c = lax.broadcasted_iota(jnp.int32, (C, C), 1)
    return (c <= r).astype(jnp.float32)


def _k1_body(x_ref, a_ref, p_ref, pq_ref, cb_ref, om_ref, wcs_ref,
             mx_ref, g2_ref, sidx_ref, carry_ref, cnt_ref):
    j = pl.program_id(1)

    @pl.when(j == 0)
    def _():
        carry_ref[...] = jnp.zeros_like(carry_ref)
        cnt_ref[0] = 0.0

    x = x_ref[0]
    x2 = jnp.concatenate([x, x], axis=1)
    tri = _tri()
    y = x2 * a_ref[...]
    cum = jnp.dot(tri, y, preferred_element_type=jnp.float32)
    s = cum * p_ref[...] + carry_ref[0:1, :] * pq_ref[...]
    carry_ref[0:1, :] = s[C - 1:C, :]
    sc = s * cb_ref[...]
    conv = sc[:, :D] + sc[:, D:]
    mx = conv + x * om_ref[...]
    mx = mx * jax.nn.sigmoid(mx)
    mx_ref[0, 0] = mx

    logit = jnp.sum(mx * wcs_ref[...], axis=1, keepdims=True)
    sel = (logit > 0.0).astype(jnp.float32)
    g2_ref[0] = jax.nn.sigmoid(logit) * sel
    csel = jnp.dot(tri, sel, preferred_element_type=jnp.float32)
    sidx_ref[0] = (csel + cnt_ref[0]) * sel
    cnt_ref[0] = cnt_ref[0] + csel[C - 1, 0]


def _run_k1(x, a, p, pq, cb, om, wcs):
    full = lambda s: pl.BlockSpec(s, lambda b, j: (0,) * len(s))
    return pl.pallas_call(
        _k1_body,
        grid=(B, NCH),
        in_specs=[
            pl.BlockSpec((1, C, D), lambda b, j: (b, j, 0)),
            full((C, 2 * D)), full((C, 2 * D)), full((C, 2 * D)),
            full((1, 2 * D)), full((1, D)), full((1, D)),
        ],
        out_specs=[
            pl.BlockSpec((1, 1, C, D), lambda b, j: (b, j, 0, 0)),
            pl.BlockSpec((1, C, 1), lambda b, j: (b, j, 0)),
            pl.BlockSpec((1, C, 1), lambda b, j: (b, j, 0)),
        ],
        out_shape=[
            jax.ShapeDtypeStruct((B, NCH, C, D), jnp.float32),
            jax.ShapeDtypeStruct((B, L, 1), jnp.float32),
            jax.ShapeDtypeStruct((B, L, 1), jnp.float32),
        ],
        scratch_shapes=[
            pltpu.VMEM((8, 2 * D), jnp.float32),
            pltpu.SMEM((1,), jnp.float32),
        ],
        compiler_params=pltpu.CompilerParams(
            dimension_semantics=("arbitrary", "arbitrary")),
    )(x, a, p, pq, cb, om, wcs)


def _k2_body(x_ref, mx_ref, wq_ref, wk_ref, wv_ref, zq_ref, zk_ref, v_ref):
    m = mx_ref[...].astype(jnp.bfloat16)
    zq_ref[...] = jnp.dot(m, wq_ref[...].astype(jnp.bfloat16),
                          preferred_element_type=jnp.float32
                          ).astype(jnp.bfloat16)
    zk_ref[...] = jnp.dot(m, wk_ref[...].astype(jnp.bfloat16),
                          preferred_element_type=jnp.float32
                          ).astype(jnp.bfloat16)
    vv = jnp.dot(x_ref[...].astype(jnp.bfloat16),
                 wv_ref[...].astype(jnp.bfloat16),
                 preferred_element_type=jnp.float32)
    v_ref[...] = (vv * jax.nn.sigmoid(vv)).astype(jnp.bfloat16)


def _run_k2(x2, mx2, wq, wk, wv):
    R = 512
    full = lambda s: pl.BlockSpec(s, lambda i: (0,) * len(s))
    return pl.pallas_call(
        _k2_body,
        grid=(BL // R,),
        in_specs=[
            pl.BlockSpec((R, D), lambda i: (i, 0)),
            pl.BlockSpec((R, D), lambda i: (i, 0)),
            full((D, Z)), full((D, Z)), full((D, D)),
        ],
        out_specs=[
            pl.BlockSpec((R, Z), lambda i: (i, 0)),
            pl.BlockSpec((R, Z), lambda i: (i, 0)),
            pl.BlockSpec((R, D), lambda i: (i, 0)),
        ],
        out_shape=[
            jax.ShapeDtypeStruct((BL, Z), jnp.bfloat16),
            jax.ShapeDtypeStruct((BL, Z), jnp.bfloat16),
            jax.ShapeDtypeStruct((BL, D), jnp.bfloat16),
        ],
        compiler_params=pltpu.CompilerParams(
            dimension_semantics=("arbitrary",)),
    )(x2, mx2, wq, wk, wv)


def _build_pos(sidx_flat, sidx_v, pos_v, b, init_val):
    pltpu.sync_copy(sidx_flat.at[pl.ds(b * L, L)], sidx_v)

    def zf(i, c):
        for k in range(GCH // 16):
            pos_v[i, pl.ds(k * 16, 16)] = jnp.full((16,), init_val,
                                                   jnp.int32)
        return c
    lax.fori_loop(0, L // GCH, zf, 0)

    base = b * L

    def scat(i, mvec):
        ranks = sidx_v[pl.ds(i * 16, 16)]
        r0 = jnp.maximum(ranks - 1, 0)
        tvec = lax.iota(jnp.int32, 16) + (i * 16 + base)
        plsc.store_scatter(pos_v,
                           [lax.shift_right_logical(r0, GCH_LOG2),
                            lax.bitwise_and(r0, GCH - 1)],
                           tvec, mask=ranks > 0)
        return jnp.maximum(mvec, ranks)
    mvec = lax.fori_loop(0, L // 16, scat, jnp.zeros((16,), jnp.int32))
    return jnp.max(mvec)


def _scgather_body(meta, sidx_flat, *rest):
    nsrc = len(meta)
    srcs = rest[:nsrc]
    outs = rest[nsrc:2 * nsrc]
    sidx_v = rest[2 * nsrc]
    pos_v = rest[2 * nsrc + 1]
    bufs = rest[2 * nsrc + 2:2 * nsrc + 2 + 2 * nsrc]
    sem = rest[-1]
    cid = lax.axis_index("c")
    sid = lax.axis_index("s")
    for t, (w, lo, hi) in enumerate(meta):
        ntiles = (hi - lo) * NC
        rpt = BL // ntiles
        nch = rpt // GCH
        src, out, b0, b1 = srcs[t], outs[t], bufs[2 * t], bufs[2 * t + 1]

        @pl.when((sid >= lo) & (sid < hi))
        def _(src=src, out=out, b0=b0, b1=b1, lo=lo, rpt=rpt, nch=nch):
            r = (sid - lo) * NC + cid
            row0 = r * rpt
            b = row0 // L
            lrow0 = row0 - b * L
            cnt = _build_pos(sidx_flat, sidx_v, pos_v, b, 0)
            bound = ((cnt + C - 1) // C) * C
            n_eff = jnp.clip((bound - lrow0 + GCH - 1) // GCH, 0, nch)
            ch0 = lrow0 // GCH

            @pl.when(n_eff > 0)
            def _():
                pltpu.async_copy(src.at[pos_v.at[ch0]], b0, sem)

                def step(i, carry):
                    for par, (cur, nxt) in enumerate(((b0, b1), (b1, b0))):
                        @pl.when(lax.rem(i, 2) == par)
                        def _(cur=cur, nxt=nxt):
                            @pl.when(i + 1 < n_eff)
                            def _():
                                pltpu.async_copy(
                                    src.at[pos_v.at[ch0 + i + 1]], nxt, sem)
                            pltpu.make_async_copy(
                                src.at[pos_v.at[ch0 + i]], cur, sem).wait()
                            pltpu.sync_copy(
                                cur, out.at[pl.ds(row0 + i * GCH, GCH)])
                    return carry
                lax.fori_loop(0, n_eff, step, 0)


def _run_scgather(sidx_flat, srcs, meta):
    mesh = plsc.VectorSubcoreMesh(core_axis_name="c", subcore_axis_name="s")
    return pl.kernel(
        functools.partial(_scgather_body, meta),
        out_type=tuple(jax.ShapeDtypeStruct((BL, w), jnp.bfloat16)
                       for (w, lo, hi) in meta),
        mesh=mesh,
        scratch_types=(
            [pltpu.VMEM((L,), jnp.int32),
             pltpu.VMEM((L // GCH, GCH), jnp.int32)]
            + [pltpu.VMEM((GCH, w), jnp.bfloat16)
               for (w, lo, hi) in meta for _ in range(2)]
            + [pltpu.SemaphoreType.DMA]
        ),
        compiler_params=pltpu.CompilerParams(needs_layout_passes=False,
                                             use_tc_tiling_on_sc=False),
    )(sidx_flat, *srcs)


def _scext_body(sidx_flat, chw_hbm, out_hbm, sidx_v, pos_v, b0, b1, sem):
    cid = lax.axis_index("c")
    sid = lax.axis_index("s")
    rpt = BL // NW
    nch = rpt // GCH
    r = sid * NC + cid
    row0 = r * rpt
    b = row0 // L
    lrow0 = row0 - b * L
    cnt = _build_pos(sidx_flat, sidx_v, pos_v, b, BL)
    n_eff = jnp.clip((cnt - lrow0 + GCH - 1) // GCH, 0, nch)
    ch0 = lrow0 // GCH

    @pl.when(n_eff > 0)
    def _():
        def step(i, carry):
            for par, (cur, nxt) in enumerate(((b0, b1), (b1, b0))):
                @pl.when(lax.rem(i, 2) == par)
                def _(cur=cur, nxt=nxt):
                    pltpu.sync_copy(
                        chw_hbm.at[pl.ds(row0 + i * GCH, GCH)], cur)

                    @pl.when(i >= 1)
                    def _():
                        pltpu.make_async_copy(
                            nxt, out_hbm.at[pos_v.at[ch0 + i]], sem).wait()
                    pltpu.async_copy(cur, out_hbm.at[pos_v.at[ch0 + i]],
                                     sem)
            return carry
        lax.fori_loop(0, n_eff, step, 0)
        pltpu.make_async_copy(b0, out_hbm.at[pos_v.at[ch0]], sem).wait()


def _run_scext(sidx_flat, chw2):
    mesh = plsc.VectorSubcoreMesh(core_axis_name="c", subcore_axis_name="s")
    return pl.kernel(
        _scext_body,
        out_type=jax.ShapeDtypeStruct((BL + 16, D), jnp.bfloat16),
        mesh=mesh,
        scratch_types=[
            pltpu.VMEM((L,), jnp.int32),
            pltpu.VMEM((L // GCH, GCH), jnp.int32),
            pltpu.VMEM((GCH, D), jnp.bfloat16),
            pltpu.VMEM((GCH, D), jnp.bfloat16),
            pltpu.SemaphoreType.DMA,
        ],
        compiler_params=pltpu.CompilerParams(needs_layout_passes=False,
                                             use_tc_tiling_on_sc=False),
    )(sidx_flat, chw2)


def _k3_body(czq_ref, czk_ref, cv_ref, bias_ref, wh_ref, counts_ref, chw_ref):
    b = pl.program_id(0)
    j = pl.program_id(1)
    q = czq_ref[0, 0]
    k = czk_ref[0, 0]
    s = lax.dot_general(q, k, (((1,), (1,)), ((), ())),
                        preferred_element_type=jnp.float32)
    s = s * (1.0 / math.sqrt(Z)) + bias_ref[...]
    col = lax.broadcasted_iota(jnp.int32, (C, C), 1) + j * C
    s = jnp.where(col < counts_ref[b], s, NEG)
    m = jnp.max(s, axis=-1, keepdims=True)
    e = jnp.exp(s - m)
    attn = (e / jnp.sum(e, axis=-1, keepdims=True)).astype(jnp.bfloat16)
    h = jnp.dot(attn, cv_ref[0, 0], preferred_element_type=jnp.float32)
    chw_ref[0, 0] = jnp.dot(h.astype(jnp.bfloat16),
                            wh_ref[...].astype(jnp.bfloat16),
                            preferred_element_type=jnp.float32
                            ).astype(jnp.bfloat16)


def _run_k3(czq, czk, cv, bias, wh, counts):
    full = lambda s: pl.BlockSpec(s, lambda b, j: (0,) * len(s))
    return pl.pallas_call(
        _k3_body,
        grid=(B, NCH),
        in_specs=[
            pl.BlockSpec((1, 1, C, Z), lambda b, j: (b, j, 0, 0)),
            pl.BlockSpec((1, 1, C, Z), lambda b, j: (b, j, 0, 0)),
            pl.BlockSpec((1, 1, C, D), lambda b, j: (b, j, 0, 0)),
            full((C, C)), full((D, D)),
            pl.BlockSpec(memory_space=pltpu.SMEM),
        ],
        out_specs=pl.BlockSpec((1, 1, C, D), lambda b, j: (b, j, 0, 0)),
        out_shape=jax.ShapeDtypeStruct((B, NCH, C, D), jnp.bfloat16),
        compiler_params=pltpu.CompilerParams(
            dimension_semantics=("arbitrary", "arbitrary")),
    )(czq, czk, cv, bias, wh, counts)


def _k4_body(x_ref, hxw_ref, g2_ref, out_ref):
    g = g2_ref[...]
    hw = hxw_ref[...].astype(jnp.float32)
    out_ref[...] = jnp.where(g > 0.0, x_ref[...] + hw * g, x_ref[...])


def _run_k4(x2, hxw, g2):
    R = 512
    return pl.pallas_call(
        _k4_body,
        grid=(BL // R,),
        in_specs=[
            pl.BlockSpec((R, D), lambda i: (i, 0)),
            pl.BlockSpec((R, D), lambda i: (i, 0)),
            pl.BlockSpec((R, 1), lambda i: (i, 0)),
        ],
        out_specs=pl.BlockSpec((R, D), lambda i: (i, 0)),
        out_shape=jax.ShapeDtypeStruct((BL, D), jnp.float32),
        compiler_params=pltpu.CompilerParams(
            dimension_semantics=("arbitrary",)),
    )(x2, hxw, g2)


def kernel(x, delta, alpha, beta, gamma, omega, temp, w_conf, w_q, w_k,
           w_v, w_h, rel_pos_bias):
    p = jax.nn.sigmoid(delta[:, :, 0])
    aa = jax.nn.sigmoid(alpha[:, :, 0])
    qd = 1.0 - p * aa
    logq = jnp.log(qd)
    cc = p * beta[:, :, 0] * gamma * math.sqrt(1.0 / N_DIM_CONST)
    t = jnp.arange(C, dtype=jnp.float32)[:, None, None]
    apow = jnp.exp(-t * logq[None])
    ppow = jnp.exp(t * logq[None])
    pq = ppow * qd[None]
    to2d = lambda z: jnp.concatenate([z[:, :, 0], z[:, :, 1]], axis=1)
    a2 = to2d(apow)
    p2 = to2d(ppow)
    pq2 = to2d(pq)
    cb = jnp.concatenate([cc[:, 0], cc[:, 1]])[None, :]
    om = omega[None, :]
    wcs = (w_conf[:, 0] / jnp.exp(temp[0]))[None, :]

    mx4, g2, sidx = _run_k1(x, a2, p2, pq2, cb, om, wcs)
    mx2 = mx4.reshape(BL, D)
    x2 = x.reshape(BL, D)
    if _STAGE == 1:
        return mx2.reshape(B, L, D)

    zq2, zk2, v2 = _run_k2(x2, mx2, w_q, w_k, w_v)
    if _STAGE == 2:
        return v2.astype(jnp.float32).reshape(B, L, D)

    sidx_i = sidx[:, :, 0].astype(jnp.int32)
    counts = jnp.max(sidx_i, axis=1)
    sidx_flat = sidx_i.reshape(BL)

    czq2, czk2, cv2 = _run_scgather(
        sidx_flat, (zq2, zk2, v2),
        ((Z, 8, 12), (Z, 12, 16), (D, 0, 8)))
    if _STAGE == 3:
        return cv2.astype(jnp.float32).reshape(B, L, D)

    off = jnp.arange(C)[:, None] - jnp.arange(C)[None, :] + C - 1
    bias = rel_pos_bias[off]
    chw = _run_k3(czq2.reshape(B, NCH, C, Z), czk2.reshape(B, NCH, C, Z),
                  cv2.reshape(B, NCH, C, D), bias, w_h, counts)
    chw2 = chw.reshape(BL, D)
    if _STAGE == 4:
        return chw2.astype(jnp.float32).reshape(B, L, D)

    hxw = _run_scext(sidx_flat, chw2)

    out2 = _run_k4(x2, hxw, g2.reshape(BL, 1))
    return out2.reshape(B, L, D)


# speedup vs baseline: 1.1789x; 1.0565x over previous
import functools
import math

import jax
import jax.numpy as jnp
from jax import lax
from jax.experimental import pallas as pl
from jax.experimental.pallas import tpu as pltpu
from jax.experimental.pallas import tpu_sc as plsc

D = 1024
Z = 256
C = 128
B = 2
L = 4096
NCH = L // C
BL = B * L
NEG = -50000.0

N_DIM_CONST = 2
_STAGE = 5

NC = 2
NS = 16
NW = NC * NS
GCH = 64
GCH_LOG2 = 6


def _tri():
    r = lax.broadcasted_iota(jnp.int32, (C, C), 0)
    c = lax.broadcasted_iota(jnp.int32, (C, C), 1)
    return (c <= r).astype(jnp.float32)


def _k1_body(x_ref, a_ref, p_ref, pq_ref, cb_ref, om_ref, wcs_ref,
             wq_ref, wk_ref, wv_ref,
             zq_ref, zk_ref, v_ref, g2_ref, sidx_ref, carry_ref, cnt_ref):
    j = pl.program_id(1)

    @pl.when(j == 0)
    def _():
        carry_ref[...] = jnp.zeros_like(carry_ref)
        cnt_ref[0] = 0.0

    x = x_ref[0]
    x2 = jnp.concatenate([x, x], axis=1)
    tri = _tri()
    y = x2 * a_ref[...]
    cum = jnp.dot(tri, y, preferred_element_type=jnp.float32)
    s = cum * p_ref[...] + carry_ref[0:1, :] * pq_ref[...]
    carry_ref[0:1, :] = s[C - 1:C, :]
    sc = s * cb_ref[...]
    conv = sc[:, :D] + sc[:, D:]
    mx = conv + x * om_ref[...]
    mx = mx * jax.nn.sigmoid(mx)

    logit = jnp.sum(mx * wcs_ref[...], axis=1, keepdims=True)
    sel = (logit > 0.0).astype(jnp.float32)
    g2_ref[0] = jax.nn.sigmoid(logit) * sel
    csel = jnp.dot(tri, sel, preferred_element_type=jnp.float32)
    sidx_ref[0] = (csel + cnt_ref[0]) * sel
    cnt_ref[0] = cnt_ref[0] + csel[C - 1, 0]

    mxb = mx.astype(jnp.bfloat16)
    zq_ref[...] = jnp.dot(mxb, wq_ref[...],
                          preferred_element_type=jnp.float32
                          ).astype(jnp.bfloat16)
    zk_ref[...] = jnp.dot(mxb, wk_ref[...],
                          preferred_element_type=jnp.float32
                          ).astype(jnp.bfloat16)
    vv = jnp.dot(x.astype(jnp.bfloat16), wv_ref[...],
                 preferred_element_type=jnp.float32)
    v_ref[...] = (vv * jax.nn.sigmoid(vv)).astype(jnp.bfloat16)


def _run_k1(x, a, p, pq, cb, om, wcs, wqb, wkb, wvb):
    full = lambda s: pl.BlockSpec(s, lambda b, j: (0,) * len(s))
    return pl.pallas_call(
        _k1_body,
        grid=(B, NCH),
        in_specs=[
            pl.BlockSpec((1, C, D), lambda b, j: (b, j, 0)),
            full((C, 2 * D)), full((C, 2 * D)), full((C, 2 * D)),
            full((1, 2 * D)), full((1, D)), full((1, D)),
            full((D, Z)), full((D, Z)), full((D, D)),
        ],
        out_specs=[
            pl.BlockSpec((C, Z), lambda b, j: (b * NCH + j, 0)),
            pl.BlockSpec((C, Z), lambda b, j: (b * NCH + j, 0)),
            pl.BlockSpec((C, D), lambda b, j: (b * NCH + j, 0)),
            pl.BlockSpec((1, C, 1), lambda b, j: (b, j, 0)),
            pl.BlockSpec((1, C, 1), lambda b, j: (b, j, 0)),
        ],
        out_shape=[
            jax.ShapeDtypeStruct((BL, Z), jnp.bfloat16),
            jax.ShapeDtypeStruct((BL, Z), jnp.bfloat16),
            jax.ShapeDtypeStruct((BL, D), jnp.bfloat16),
            jax.ShapeDtypeStruct((B, L, 1), jnp.float32),
            jax.ShapeDtypeStruct((B, L, 1), jnp.float32),
        ],
        scratch_shapes=[
            pltpu.VMEM((8, 2 * D), jnp.float32),
            pltpu.SMEM((1,), jnp.float32),
        ],
        compiler_params=pltpu.CompilerParams(
            dimension_semantics=("arbitrary", "arbitrary")),
    )(x, a, p, pq, cb, om, wcs, wqb, wkb, wvb)


def _build_pos(sidx_flat, sidx_v, pos_v, b, init_val):
    pltpu.sync_copy(sidx_flat.at[pl.ds(b * L, L)], sidx_v)

    def zf(i, c):
        for k in range(GCH // 16):
            pos_v[i, pl.ds(k * 16, 16)] = jnp.full((16,), init_val,
                                                   jnp.int32)
        return c
    lax.fori_loop(0, L // GCH, zf, 0)

    base = b * L

    def scat(i, mvec):
        ranks = sidx_v[pl.ds(i * 16, 16)]
        r0 = jnp.maximum(ranks - 1, 0)
        tvec = lax.iota(jnp.int32, 16) + (i * 16 + base)
        plsc.store_scatter(pos_v,
                           [lax.shift_right_logical(r0, GCH_LOG2),
                            lax.bitwise_and(r0, GCH - 1)],
                           tvec, mask=ranks > 0)
        return jnp.maximum(mvec, ranks)
    mvec = lax.fori_loop(0, L // 16, scat, jnp.zeros((16,), jnp.int32))
    return jnp.max(mvec)


def _scgather_body(meta, sidx_flat, *rest):
    nsrc = len(meta)
    srcs = rest[:nsrc]
    outs = rest[nsrc:2 * nsrc]
    sidx_v = rest[2 * nsrc]
    pos_v = rest[2 * nsrc + 1]
    bufs = rest[2 * nsrc + 2:2 * nsrc + 2 + 2 * nsrc]
    sem = rest[-1]
    cid = lax.axis_index("c")
    sid = lax.axis_index("s")
    for t, (w, lo, hi) in enumerate(meta):
        ntiles = (hi - lo) * NC
        rpt = BL // ntiles
        nch = rpt // GCH
        src, out, b0, b1 = srcs[t], outs[t], bufs[2 * t], bufs[2 * t + 1]

        @pl.when((sid >= lo) & (sid < hi))
        def _(src=src, out=out, b0=b0, b1=b1, lo=lo, rpt=rpt, nch=nch):
            r = (sid - lo) * NC + cid
            row0 = r * rpt
            b = row0 // L
            lrow0 = row0 - b * L
            cnt = _build_pos(sidx_flat, sidx_v, pos_v, b, 0)
            bound = ((cnt + C - 1) // C) * C
            n_eff = jnp.clip((bound - lrow0 + GCH - 1) // GCH, 0, nch)
            ch0 = lrow0 // GCH

            @pl.when(n_eff > 0)
            def _():
                pltpu.async_copy(src.at[pos_v.at[ch0]], b0, sem)

                def step(i, carry):
                    for par, (cur, nxt) in enumerate(((b0, b1), (b1, b0))):
                        @pl.when(lax.rem(i, 2) == par)
                        def _(cur=cur, nxt=nxt):
                            @pl.when(i + 1 < n_eff)
                            def _():
                                pltpu.async_copy(
                                    src.at[pos_v.at[ch0 + i + 1]], nxt, sem)
                            pltpu.make_async_copy(
                                src.at[pos_v.at[ch0 + i]], cur, sem).wait()
                            pltpu.sync_copy(
                                cur, out.at[pl.ds(row0 + i * GCH, GCH)])
                    return carry
                lax.fori_loop(0, n_eff, step, 0)


def _run_scgather(sidx_flat, srcs, meta):
    mesh = plsc.VectorSubcoreMesh(core_axis_name="c", subcore_axis_name="s")
    return pl.kernel(
        functools.partial(_scgather_body, meta),
        out_type=tuple(jax.ShapeDtypeStruct((BL, w), jnp.bfloat16)
                       for (w, lo, hi) in meta),
        mesh=mesh,
        scratch_types=(
            [pltpu.VMEM((L,), jnp.int32),
             pltpu.VMEM((L // GCH, GCH), jnp.int32)]
            + [pltpu.VMEM((GCH, w), jnp.bfloat16)
               for (w, lo, hi) in meta for _ in range(2)]
            + [pltpu.SemaphoreType.DMA]
        ),
        compiler_params=pltpu.CompilerParams(needs_layout_passes=False,
                                             use_tc_tiling_on_sc=False),
    )(sidx_flat, *srcs)


def _scext_body(sidx_flat, chw_hbm, out_hbm, sidx_v, pos_v, b0, b1, sem):
    cid = lax.axis_index("c")
    sid = lax.axis_index("s")
    rpt = BL // NW
    nch = rpt // GCH
    r = sid * NC + cid
    row0 = r * rpt
    b = row0 // L
    lrow0 = row0 - b * L
    cnt = _build_pos(sidx_flat, sidx_v, pos_v, b, BL)
    n_eff = jnp.clip((cnt - lrow0 + GCH - 1) // GCH, 0, nch)
    ch0 = lrow0 // GCH

    @pl.when(n_eff > 0)
    def _():
        def step(i, carry):
            for par, (cur, nxt) in enumerate(((b0, b1), (b1, b0))):
                @pl.when(lax.rem(i, 2) == par)
                def _(cur=cur, nxt=nxt):
                    pltpu.sync_copy(
                        chw_hbm.at[pl.ds(row0 + i * GCH, GCH)], cur)

                    @pl.when(i >= 1)
                    def _():
                        pltpu.make_async_copy(
                            nxt, out_hbm.at[pos_v.at[ch0 + i]], sem).wait()
                    pltpu.async_copy(cur, out_hbm.at[pos_v.at[ch0 + i]],
                                     sem)
            return carry
        lax.fori_loop(0, n_eff, step, 0)
        pltpu.make_async_copy(b0, out_hbm.at[pos_v.at[ch0]], sem).wait()


def _run_scext(sidx_flat, chw2):
    mesh = plsc.VectorSubcoreMesh(core_axis_name="c", subcore_axis_name="s")
    return pl.kernel(
        _scext_body,
        out_type=jax.ShapeDtypeStruct((BL + 16, D), jnp.bfloat16),
        mesh=mesh,
        scratch_types=[
            pltpu.VMEM((L,), jnp.int32),
            pltpu.VMEM((L // GCH, GCH), jnp.int32),
            pltpu.VMEM((GCH, D), jnp.bfloat16),
            pltpu.VMEM((GCH, D), jnp.bfloat16),
            pltpu.SemaphoreType.DMA,
        ],
        compiler_params=pltpu.CompilerParams(needs_layout_passes=False,
                                             use_tc_tiling_on_sc=False),
    )(sidx_flat, chw2)


def _k3_body(czq_ref, czk_ref, cv_ref, bias_ref, wh_ref, counts_ref, chw_ref):
    b = pl.program_id(0)
    j = pl.program_id(1)
    cnt = counts_ref[b]

    @pl.when(j * C < cnt)
    def _():
        q = czq_ref[0, 0]
        k = czk_ref[0, 0]
        s = lax.dot_general(q, k, (((1,), (1,)), ((), ())),
                            preferred_element_type=jnp.float32)
        s = s * (1.0 / math.sqrt(Z)) + bias_ref[...]
        col = lax.broadcasted_iota(jnp.int32, (C, C), 1) + j * C
        s = jnp.where(col < cnt, s, NEG)
        m = jnp.max(s, axis=-1, keepdims=True)
        e = jnp.exp(s - m)
        attn = (e / jnp.sum(e, axis=-1, keepdims=True)).astype(jnp.bfloat16)
        h = jnp.dot(attn, cv_ref[0, 0], preferred_element_type=jnp.float32)
        chw_ref[0, 0] = jnp.dot(h.astype(jnp.bfloat16),
                                wh_ref[...].astype(jnp.bfloat16),
                                preferred_element_type=jnp.float32
                                ).astype(jnp.bfloat16)


def _run_k3(czq, czk, cv, bias, wh, counts):
    full = lambda s: pl.BlockSpec(s, lambda b, j: (0,) * len(s))
    return pl.pallas_call(
        _k3_body,
        grid=(B, NCH),
        in_specs=[
            pl.BlockSpec((1, 1, C, Z), lambda b, j: (b, j, 0, 0)),
            pl.BlockSpec((1, 1, C, Z), lambda b, j: (b, j, 0, 0)),
            pl.BlockSpec((1, 1, C, D), lambda b, j: (b, j, 0, 0)),
            full((C, C)), full((D, D)),
            pl.BlockSpec(memory_space=pltpu.SMEM),
        ],
        out_specs=pl.BlockSpec((1, 1, C, D), lambda b, j: (b, j, 0, 0)),
        out_shape=jax.ShapeDtypeStruct((B, NCH, C, D), jnp.bfloat16),
        compiler_params=pltpu.CompilerParams(
            dimension_semantics=("arbitrary", "arbitrary")),
    )(czq, czk, cv, bias, wh, counts)


def _k4_body(x_ref, hxw_ref, g2_ref, out_ref):
    g = g2_ref[...]
    hw = hxw_ref[...].astype(jnp.float32)
    out_ref[...] = jnp.where(g > 0.0, x_ref[...] + hw * g, x_ref[...])


def _run_k4(x2, hxw, g2):
    R = 512
    return pl.pallas_call(
        _k4_body,
        grid=(BL // R,),
        in_specs=[
            pl.BlockSpec((R, D), lambda i: (i, 0)),
            pl.BlockSpec((R, D), lambda i: (i, 0)),
            pl.BlockSpec((R, 1), lambda i: (i, 0)),
        ],
        out_specs=pl.BlockSpec((R, D), lambda i: (i, 0)),
        out_shape=jax.ShapeDtypeStruct((BL, D), jnp.float32),
        compiler_params=pltpu.CompilerParams(
            dimension_semantics=("arbitrary",)),
    )(x2, hxw, g2)


def kernel(x, delta, alpha, beta, gamma, omega, temp, w_conf, w_q, w_k,
           w_v, w_h, rel_pos_bias):
    p = jax.nn.sigmoid(delta[:, :, 0])
    aa = jax.nn.sigmoid(alpha[:, :, 0])
    qd = 1.0 - p * aa
    logq = jnp.log(qd)
    cc = p * beta[:, :, 0] * gamma * math.sqrt(1.0 / N_DIM_CONST)
    t = jnp.arange(C, dtype=jnp.float32)[:, None, None]
    apow = jnp.exp(-t * logq[None])
    ppow = jnp.exp(t * logq[None])
    pq = ppow * qd[None]
    to2d = lambda z: jnp.concatenate([z[:, :, 0], z[:, :, 1]], axis=1)
    a2 = to2d(apow)
    p2 = to2d(ppow)
    pq2 = to2d(pq)
    cb = jnp.concatenate([cc[:, 0], cc[:, 1]])[None, :]
    om = omega[None, :]
    wcs = (w_conf[:, 0] / jnp.exp(temp[0]))[None, :]

    zq2, zk2, v2, g2, sidx = _run_k1(
        x, a2, p2, pq2, cb, om, wcs,
        w_q.astype(jnp.bfloat16), w_k.astype(jnp.bfloat16),
        w_v.astype(jnp.bfloat16))
    x2 = x.reshape(BL, D)
    if _STAGE == 2:
        return v2.astype(jnp.float32).reshape(B, L, D)

    sidx_i = sidx[:, :, 0].astype(jnp.int32)
    counts = jnp.max(sidx_i, axis=1)
    sidx_flat = sidx_i.reshape(BL)

    czq2, czk2, cv2 = _run_scgather(
        sidx_flat, (zq2, zk2, v2),
        ((Z, 8, 12), (Z, 12, 16), (D, 0, 8)))
    if _STAGE == 3:
        return cv2.astype(jnp.float32).reshape(B, L, D)

    off = jnp.arange(C)[:, None] - jnp.arange(C)[None, :] + C - 1
    bias = rel_pos_bias[off]
    chw = _run_k3(czq2.reshape(B, NCH, C, Z), czk2.reshape(B, NCH, C, Z),
                  cv2.reshape(B, NCH, C, D), bias, w_h, counts)
    chw2 = chw.reshape(BL, D)
    if _STAGE == 4:
        return chw2.astype(jnp.float32).reshape(B, L, D)

    hxw = _run_scext(sidx_flat, chw2)

    out2 = _run_k4(x2, hxw, g2.reshape(BL, 1))
    return out2.reshape(B, L, D)


# speedup vs baseline: 1.5296x; 1.2975x over previous
import functools
import math

import jax
import jax.numpy as jnp
from jax import lax
from jax.experimental import pallas as pl
from jax.experimental.pallas import tpu as pltpu
from jax.experimental.pallas import tpu_sc as plsc

D = 1024
Z = 256
C = 128
B = 2
L = 4096
NCH = L // C
BL = B * L
NEG = -50000.0

N_DIM_CONST = 2
_STAGE = 5

NC = 2
NS = 16
NW = NC * NS
GCH = 64
GCH_LOG2 = 6


def _tri():
    r = lax.broadcasted_iota(jnp.int32, (C, C), 0)
    c = lax.broadcasted_iota(jnp.int32, (C, C), 1)
    return (c <= r).astype(jnp.float32)


def _k1_body(x_ref, a_ref, p_ref, pq_ref, cb_ref, om_ref, wcs_ref,
             wq_ref, wk_ref, wv_ref,
             zq_ref, zk_ref, v_ref, g2_ref, sidx_ref, carry_ref, cnt_ref):
    j = pl.program_id(1)

    @pl.when(j == 0)
    def _():
        carry_ref[...] = jnp.zeros_like(carry_ref)
        cnt_ref[0] = 0.0

    x = x_ref[0]
    x2 = jnp.concatenate([x, x], axis=1)
    tri = _tri()
    y = x2 * a_ref[...]
    cum = jnp.dot(tri, y, preferred_element_type=jnp.float32)
    s = cum * p_ref[...] + carry_ref[0:1, :] * pq_ref[...]
    carry_ref[0:1, :] = s[C - 1:C, :]
    sc = s * cb_ref[...]
    conv = sc[:, :D] + sc[:, D:]
    mx = conv + x * om_ref[...]
    mx = mx * jax.nn.sigmoid(mx)

    logit = jnp.sum(mx * wcs_ref[...], axis=1, keepdims=True)
    sel = (logit > 0.0).astype(jnp.float32)
    g2_ref[0] = jax.nn.sigmoid(logit) * sel
    csel = jnp.dot(tri, sel, preferred_element_type=jnp.float32)
    sidx_ref[0] = (csel + cnt_ref[0]) * sel
    cnt_ref[0] = cnt_ref[0] + csel[C - 1, 0]

    mxb = mx.astype(jnp.bfloat16)
    zq_ref[...] = jnp.dot(mxb, wq_ref[...],
                          preferred_element_type=jnp.float32
                          ).astype(jnp.bfloat16)
    zk_ref[...] = jnp.dot(mxb, wk_ref[...],
                          preferred_element_type=jnp.float32
                          ).astype(jnp.bfloat16)
    vv = jnp.dot(x.astype(jnp.bfloat16), wv_ref[...],
                 preferred_element_type=jnp.float32)
    v_ref[...] = (vv * jax.nn.sigmoid(vv)).astype(jnp.bfloat16)


def _run_k1(x, a, p, pq, cb, om, wcs, wqb, wkb, wvb):
    full = lambda s: pl.BlockSpec(s, lambda b, j: (0,) * len(s))
    return pl.pallas_call(
        _k1_body,
        grid=(B, NCH),
        in_specs=[
            pl.BlockSpec((1, C, D), lambda b, j: (b, j, 0)),
            full((C, 2 * D)), full((C, 2 * D)), full((C, 2 * D)),
            full((1, 2 * D)), full((1, D)), full((1, D)),
            full((D, Z)), full((D, Z)), full((D, D)),
        ],
        out_specs=[
            pl.BlockSpec((C, Z), lambda b, j: (b * NCH + j, 0)),
            pl.BlockSpec((C, Z), lambda b, j: (b * NCH + j, 0)),
            pl.BlockSpec((C, D), lambda b, j: (b * NCH + j, 0)),
            pl.BlockSpec((1, C, 1), lambda b, j: (b, j, 0)),
            pl.BlockSpec((1, C, 1), lambda b, j: (b, j, 0)),
        ],
        out_shape=[
            jax.ShapeDtypeStruct((BL, Z), jnp.bfloat16),
            jax.ShapeDtypeStruct((BL, Z), jnp.bfloat16),
            jax.ShapeDtypeStruct((BL, D), jnp.bfloat16),
            jax.ShapeDtypeStruct((B, L, 1), jnp.float32),
            jax.ShapeDtypeStruct((B, L, 1), jnp.float32),
        ],
        scratch_shapes=[
            pltpu.VMEM((8, 2 * D), jnp.float32),
            pltpu.SMEM((1,), jnp.float32),
        ],
        compiler_params=pltpu.CompilerParams(
            dimension_semantics=("arbitrary", "arbitrary")),
    )(x, a, p, pq, cb, om, wcs, wqb, wkb, wvb)


def _build_pos(sidx_flat, sidx_v, pos_v, b, init_val):
    pltpu.sync_copy(sidx_flat.at[pl.ds(b * L, L)], sidx_v)

    def zf(i, c):
        for k in range(GCH // 16):
            pos_v[i, pl.ds(k * 16, 16)] = jnp.full((16,), init_val,
                                                   jnp.int32)
        return c
    lax.fori_loop(0, L // GCH, zf, 0)

    base = b * L

    def scat(i, mvec):
        ranks = sidx_v[pl.ds(i * 16, 16)]
        r0 = jnp.maximum(ranks - 1, 0)
        tvec = lax.iota(jnp.int32, 16) + (i * 16 + base)
        plsc.store_scatter(pos_v,
                           [lax.shift_right_logical(r0, GCH_LOG2),
                            lax.bitwise_and(r0, GCH - 1)],
                           tvec, mask=ranks > 0)
        return jnp.maximum(mvec, ranks)
    mvec = lax.fori_loop(0, L // 16, scat, jnp.zeros((16,), jnp.int32))
    return jnp.max(mvec)


def _scgather_body(meta, sidx_flat, *rest):
    nsrc = len(meta)
    srcs = rest[:nsrc]
    outs = rest[nsrc:2 * nsrc]
    sidx_v = rest[2 * nsrc]
    pos_v = rest[2 * nsrc + 1]
    bufs = rest[2 * nsrc + 2:2 * nsrc + 2 + 2 * nsrc]
    sem = rest[-1]
    cid = lax.axis_index("c")
    sid = lax.axis_index("s")
    for t, (w, lo, hi) in enumerate(meta):
        ntiles = (hi - lo) * NC
        rpt = BL // ntiles
        nch = rpt // GCH
        src, out, b0, b1 = srcs[t], outs[t], bufs[2 * t], bufs[2 * t + 1]

        @pl.when((sid >= lo) & (sid < hi))
        def _(src=src, out=out, b0=b0, b1=b1, lo=lo, rpt=rpt, nch=nch):
            r = (sid - lo) * NC + cid
            row0 = r * rpt
            b = row0 // L
            lrow0 = row0 - b * L
            cnt = _build_pos(sidx_flat, sidx_v, pos_v, b, 0)
            bound = ((cnt + C - 1) // C) * C
            n_eff = jnp.clip((bound - lrow0 + GCH - 1) // GCH, 0, nch)
            ch0 = lrow0 // GCH

            @pl.when(n_eff > 0)
            def _():
                pltpu.async_copy(src.at[pos_v.at[ch0]], b0, sem)

                def step(i, carry):
                    for par, (cur, nxt) in enumerate(((b0, b1), (b1, b0))):
                        @pl.when(lax.rem(i, 2) == par)
                        def _(cur=cur, nxt=nxt):
                            @pl.when(i + 1 < n_eff)
                            def _():
                                pltpu.async_copy(
                                    src.at[pos_v.at[ch0 + i + 1]], nxt, sem)
                            pltpu.make_async_copy(
                                src.at[pos_v.at[ch0 + i]], cur, sem).wait()
                            pltpu.sync_copy(
                                cur, out.at[pl.ds(row0 + i * GCH, GCH)])
                    return carry
                lax.fori_loop(0, n_eff, step, 0)


def _run_scgather(sidx_flat, srcs, meta):
    mesh = plsc.VectorSubcoreMesh(core_axis_name="c", subcore_axis_name="s")
    return pl.kernel(
        functools.partial(_scgather_body, meta),
        out_type=tuple(jax.ShapeDtypeStruct((BL, w), jnp.bfloat16)
                       for (w, lo, hi) in meta),
        mesh=mesh,
        scratch_types=(
            [pltpu.VMEM((L,), jnp.int32),
             pltpu.VMEM((L // GCH, GCH), jnp.int32)]
            + [pltpu.VMEM((GCH, w), jnp.bfloat16)
               for (w, lo, hi) in meta for _ in range(2)]
            + [pltpu.SemaphoreType.DMA]
        ),
        compiler_params=pltpu.CompilerParams(needs_layout_passes=False,
                                             use_tc_tiling_on_sc=False),
    )(sidx_flat, *srcs)


def _scext_body(sidx_flat, chw_hbm, out_hbm, sidx_v, pos_v, b0, b1, sem):
    cid = lax.axis_index("c")
    sid = lax.axis_index("s")
    rpt = BL // NW
    nch = rpt // GCH
    r = sid * NC + cid
    row0 = r * rpt
    b = row0 // L
    lrow0 = row0 - b * L
    cnt = _build_pos(sidx_flat, sidx_v, pos_v, b, BL)
    n_eff = jnp.clip((cnt - lrow0 + GCH - 1) // GCH, 0, nch)
    ch0 = lrow0 // GCH

    @pl.when(n_eff > 0)
    def _():
        def step(i, carry):
            for par, (cur, nxt) in enumerate(((b0, b1), (b1, b0))):
                @pl.when(lax.rem(i, 2) == par)
                def _(cur=cur, nxt=nxt):
                    pltpu.sync_copy(
                        chw_hbm.at[pl.ds(row0 + i * GCH, GCH)], cur)

                    @pl.when(i >= 1)
                    def _():
                        pltpu.make_async_copy(
                            nxt, out_hbm.at[pos_v.at[ch0 + i]], sem).wait()
                    pltpu.async_copy(cur, out_hbm.at[pos_v.at[ch0 + i]],
                                     sem)
            return carry
        lax.fori_loop(0, n_eff, step, 0)
        pltpu.make_async_copy(b0, out_hbm.at[pos_v.at[ch0]], sem).wait()


def _run_scext(sidx_flat, chw2):
    mesh = plsc.VectorSubcoreMesh(core_axis_name="c", subcore_axis_name="s")
    return pl.kernel(
        _scext_body,
        out_type=jax.ShapeDtypeStruct((BL + 16, D), jnp.bfloat16),
        mesh=mesh,
        scratch_types=[
            pltpu.VMEM((L,), jnp.int32),
            pltpu.VMEM((L // GCH, GCH), jnp.int32),
            pltpu.VMEM((GCH, D), jnp.bfloat16),
            pltpu.VMEM((GCH, D), jnp.bfloat16),
            pltpu.SemaphoreType.DMA,
        ],
        compiler_params=pltpu.CompilerParams(needs_layout_passes=False,
                                             use_tc_tiling_on_sc=False),
    )(sidx_flat, chw2)


def _k3_body(czq_ref, czk_ref, cv_ref, bias_ref, whb_ref, counts_ref,
             chw_ref):
    b = pl.program_id(0)
    j = pl.program_id(1)
    cnt = counts_ref[b]

    @pl.when(j * C < cnt)
    def _():
        q = czq_ref[...]
        k = czk_ref[...]
        s = lax.dot_general(q, k, (((1,), (1,)), ((), ())),
                            preferred_element_type=jnp.float32)
        s = s * (1.0 / math.sqrt(Z)) + bias_ref[...]
        col = lax.broadcasted_iota(jnp.int32, (C, C), 1) + j * C
        s = jnp.where(col < cnt, s, NEG)
        m = jnp.max(s, axis=-1, keepdims=True)
        e = jnp.exp(s - m)
        attn = (e / jnp.sum(e, axis=-1, keepdims=True)).astype(jnp.bfloat16)
        h = jnp.dot(attn, cv_ref[...], preferred_element_type=jnp.float32)
        chw_ref[...] = jnp.dot(h.astype(jnp.bfloat16), whb_ref[...],
                               preferred_element_type=jnp.float32
                               ).astype(jnp.bfloat16)


def _run_k3(czq, czk, cv, bias, whb, counts):
    full = lambda s: pl.BlockSpec(s, lambda b, j: (0,) * len(s))
    return pl.pallas_call(
        _k3_body,
        grid=(B, NCH),
        in_specs=[
            pl.BlockSpec((C, Z), lambda b, j: (b * NCH + j, 0)),
            pl.BlockSpec((C, Z), lambda b, j: (b * NCH + j, 0)),
            pl.BlockSpec((C, D), lambda b, j: (b * NCH + j, 0)),
            full((C, C)), full((D, D)),
            pl.BlockSpec(memory_space=pltpu.SMEM),
        ],
        out_specs=pl.BlockSpec((C, D), lambda b, j: (b * NCH + j, 0)),
        out_shape=jax.ShapeDtypeStruct((BL, D), jnp.bfloat16),
        compiler_params=pltpu.CompilerParams(
            dimension_semantics=("arbitrary", "arbitrary")),
    )(czq, czk, cv, bias, whb, counts)


def _k4_body(x_ref, hxw_ref, g2_ref, out_ref):
    g = g2_ref[0]
    hw = hxw_ref[...].astype(jnp.float32)
    out_ref[0] = jnp.where(g > 0.0, x_ref[0] + hw * g, x_ref[0])


def _run_k4(x, hxw, g2):
    R = 512
    NB = L // R
    return pl.pallas_call(
        _k4_body,
        grid=(B, NB),
        in_specs=[
            pl.BlockSpec((1, R, D), lambda b, i: (b, i, 0)),
            pl.BlockSpec((R, D), lambda b, i: (b * NB + i, 0)),
            pl.BlockSpec((1, R, 1), lambda b, i: (b, i, 0)),
        ],
        out_specs=pl.BlockSpec((1, R, D), lambda b, i: (b, i, 0)),
        out_shape=jax.ShapeDtypeStruct((B, L, D), jnp.float32),
        compiler_params=pltpu.CompilerParams(
            dimension_semantics=("arbitrary", "arbitrary")),
    )(x, hxw, g2)


def kernel(x, delta, alpha, beta, gamma, omega, temp, w_conf, w_q, w_k,
           w_v, w_h, rel_pos_bias):
    p = jax.nn.sigmoid(delta[:, :, 0])
    aa = jax.nn.sigmoid(alpha[:, :, 0])
    qd = 1.0 - p * aa
    logq = jnp.log(qd)
    cc = p * beta[:, :, 0] * gamma * math.sqrt(1.0 / N_DIM_CONST)
    t = jnp.arange(C, dtype=jnp.float32)[:, None, None]
    apow = jnp.exp(-t * logq[None])
    ppow = jnp.exp(t * logq[None])
    pq = ppow * qd[None]
    to2d = lambda z: jnp.concatenate([z[:, :, 0], z[:, :, 1]], axis=1)
    a2 = to2d(apow)
    p2 = to2d(ppow)
    pq2 = to2d(pq)
    cb = jnp.concatenate([cc[:, 0], cc[:, 1]])[None, :]
    om = omega[None, :]
    wcs = (w_conf[:, 0] / jnp.exp(temp[0]))[None, :]

    zq2, zk2, v2, g2, sidx = _run_k1(
        x, a2, p2, pq2, cb, om, wcs,
        w_q.astype(jnp.bfloat16), w_k.astype(jnp.bfloat16),
        w_v.astype(jnp.bfloat16))

    sidx_i = sidx[:, :, 0].astype(jnp.int32)
    counts = jnp.max(sidx_i, axis=1)
    sidx_flat = sidx_i.reshape(BL)

    czq2, czk2, cv2 = _run_scgather(
        sidx_flat, (zq2, zk2, v2),
        ((Z, 8, 12), (Z, 12, 16), (D, 0, 8)))

    u3 = jnp.concatenate([rel_pos_bias[::-1], jnp.zeros((1,), jnp.float32)])
    bias = jnp.tile(u3, C)[:C * (2 * C - 1)].reshape(C, 2 * C - 1)
    bias = bias[:, C - 1:]
    chw2 = _run_k3(czq2, czk2, cv2, bias, w_h.astype(jnp.bfloat16), counts)

    hxw = _run_scext(sidx_flat, chw2)

    return _run_k4(x, hxw, g2)


# speedup vs baseline: 1.7029x; 1.1133x over previous
import functools
import math

import jax
import jax.numpy as jnp
from jax import lax
from jax.experimental import pallas as pl
from jax.experimental.pallas import tpu as pltpu
from jax.experimental.pallas import tpu_sc as plsc

D = 1024
Z = 256
C = 128
B = 2
L = 4096
NCH = L // C
BL = B * L
NEG = -50000.0

N_DIM_CONST = 2
_STAGE = 5

NC = 2
NS = 16
NW = NC * NS
GCH = 64
GCH_LOG2 = 6
ECH = 32
ECH_LOG2 = 5


def _tri():
    r = lax.broadcasted_iota(jnp.int32, (C, C), 0)
    c = lax.broadcasted_iota(jnp.int32, (C, C), 1)
    return (c <= r).astype(jnp.float32)


def _k1_body(x_ref, a_ref, p_ref, pq_ref, cb_ref, om_ref, wcs_ref,
             wq_ref, wk_ref, wv_ref,
             zq_ref, zk_ref, v_ref, g2_ref, sidx_ref, carry_ref, cnt_ref):
    j = pl.program_id(1)

    @pl.when(j == 0)
    def _():
        carry_ref[...] = jnp.zeros_like(carry_ref)
        cnt_ref[0] = 0.0

    x = x_ref[0]
    x2 = jnp.concatenate([x, x], axis=1)
    tri = _tri()
    y = x2 * a_ref[...]
    cum = jnp.dot(tri, y, preferred_element_type=jnp.float32)
    s = cum * p_ref[...] + carry_ref[0:1, :] * pq_ref[...]
    carry_ref[0:1, :] = s[C - 1:C, :]
    sc = s * cb_ref[...]
    conv = sc[:, :D] + sc[:, D:]
    mx = conv + x * om_ref[...]
    mx = mx * jax.nn.sigmoid(mx)

    logit = jnp.sum(mx * wcs_ref[...], axis=1, keepdims=True)
    sel = (logit > 0.0).astype(jnp.float32)
    g2_ref[0] = jax.nn.sigmoid(logit) * sel
    csel = jnp.dot(tri, sel, preferred_element_type=jnp.float32)
    sidx_ref[0] = (csel + cnt_ref[0]) * sel
    cnt_ref[0] = cnt_ref[0] + csel[C - 1, 0]

    mxb = mx.astype(jnp.bfloat16)
    zq_ref[...] = jnp.dot(mxb, wq_ref[...],
                          preferred_element_type=jnp.float32
                          ).astype(jnp.bfloat16)
    zk_ref[...] = jnp.dot(mxb, wk_ref[...],
                          preferred_element_type=jnp.float32
                          ).astype(jnp.bfloat16)
    vv = jnp.dot(x.astype(jnp.bfloat16), wv_ref[...],
                 preferred_element_type=jnp.float32)
    v_ref[...] = (vv * jax.nn.sigmoid(vv)).astype(jnp.bfloat16)


def _run_k1(x, a, p, pq, cb, om, wcs, wqb, wkb, wvb):
    full = lambda s: pl.BlockSpec(s, lambda b, j: (0,) * len(s))
    return pl.pallas_call(
        _k1_body,
        grid=(B, NCH),
        in_specs=[
            pl.BlockSpec((1, C, D), lambda b, j: (b, j, 0)),
            full((C, 2 * D)), full((C, 2 * D)), full((C, 2 * D)),
            full((1, 2 * D)), full((1, D)), full((1, D)),
            full((D, Z)), full((D, Z)), full((D, D)),
        ],
        out_specs=[
            pl.BlockSpec((C, Z), lambda b, j: (b * NCH + j, 0)),
            pl.BlockSpec((C, Z), lambda b, j: (b * NCH + j, 0)),
            pl.BlockSpec((C, D), lambda b, j: (b * NCH + j, 0)),
            pl.BlockSpec((1, C, 1), lambda b, j: (b, j, 0)),
            pl.BlockSpec((1, C, 1), lambda b, j: (b, j, 0)),
        ],
        out_shape=[
            jax.ShapeDtypeStruct((BL, Z), jnp.bfloat16),
            jax.ShapeDtypeStruct((BL, Z), jnp.bfloat16),
            jax.ShapeDtypeStruct((BL, D), jnp.bfloat16),
            jax.ShapeDtypeStruct((B, L, 1), jnp.float32),
            jax.ShapeDtypeStruct((B, L, 1), jnp.float32),
        ],
        scratch_shapes=[
            pltpu.VMEM((8, 2 * D), jnp.float32),
            pltpu.SMEM((1,), jnp.float32),
        ],
        compiler_params=pltpu.CompilerParams(
            dimension_semantics=("arbitrary", "arbitrary")),
    )(x, a, p, pq, cb, om, wcs, wqb, wkb, wvb)


def _build_pos(sidx_flat, sidx_v, pos_v, b, init_val, lg=GCH_LOG2):
    w = 1 << lg
    pltpu.sync_copy(sidx_flat.at[pl.ds(b * L, L)], sidx_v)

    def zf(i, c):
        for k in range(w // 16):
            pos_v[i, pl.ds(k * 16, 16)] = jnp.full((16,), init_val,
                                                   jnp.int32)
        return c
    lax.fori_loop(0, L // w, zf, 0)

    base = b * L

    def scat(i, mvec):
        ranks = sidx_v[pl.ds(i * 16, 16)]
        r0 = jnp.maximum(ranks - 1, 0)
        tvec = lax.iota(jnp.int32, 16) + (i * 16 + base)
        plsc.store_scatter(pos_v,
                           [lax.shift_right_logical(r0, lg),
                            lax.bitwise_and(r0, (1 << lg) - 1)],
                           tvec, mask=ranks > 0)
        return jnp.maximum(mvec, ranks)
    mvec = lax.fori_loop(0, L // 16, scat, jnp.zeros((16,), jnp.int32))
    return jnp.max(mvec)


def _scgather_body(meta, sidx_flat, *rest):
    nsrc = len(meta)
    srcs = rest[:nsrc]
    outs = rest[nsrc:2 * nsrc]
    sidx_v = rest[2 * nsrc]
    pos_v = rest[2 * nsrc + 1]
    bufs = rest[2 * nsrc + 2:2 * nsrc + 2 + 2 * nsrc]
    sem = rest[-1]
    cid = lax.axis_index("c")
    sid = lax.axis_index("s")
    for t, (w, lo, hi) in enumerate(meta):
        ntiles = (hi - lo) * NC
        rpt = BL // ntiles
        nch = rpt // GCH
        src, out, b0, b1 = srcs[t], outs[t], bufs[2 * t], bufs[2 * t + 1]

        @pl.when((sid >= lo) & (sid < hi))
        def _(src=src, out=out, b0=b0, b1=b1, lo=lo, rpt=rpt, nch=nch):
            r = (sid - lo) * NC + cid
            row0 = r * rpt
            b = row0 // L
            lrow0 = row0 - b * L
            cnt = _build_pos(sidx_flat, sidx_v, pos_v, b, 0)
            bound = ((cnt + C - 1) // C) * C
            n_eff = jnp.clip((bound - lrow0 + GCH - 1) // GCH, 0, nch)
            ch0 = lrow0 // GCH

            @pl.when(n_eff > 0)
            def _():
                pltpu.async_copy(src.at[pos_v.at[ch0]], b0, sem)

                def step(i, carry):
                    for par, (cur, nxt) in enumerate(((b0, b1), (b1, b0))):
                        @pl.when(lax.rem(i, 2) == par)
                        def _(cur=cur, nxt=nxt):
                            @pl.when(i + 1 < n_eff)
                            def _():
                                pltpu.async_copy(
                                    src.at[pos_v.at[ch0 + i + 1]], nxt, sem)
                            pltpu.make_async_copy(
                                src.at[pos_v.at[ch0 + i]], cur, sem).wait()
                            pltpu.sync_copy(
                                cur, out.at[pl.ds(row0 + i * GCH, GCH)])
                    return carry
                lax.fori_loop(0, n_eff, step, 0)


def _run_scgather(sidx_flat, srcs, meta):
    mesh = plsc.VectorSubcoreMesh(core_axis_name="c", subcore_axis_name="s")
    return pl.kernel(
        functools.partial(_scgather_body, meta),
        out_type=tuple(jax.ShapeDtypeStruct((BL, w), jnp.bfloat16)
                       for (w, lo, hi) in meta),
        mesh=mesh,
        scratch_types=(
            [pltpu.VMEM((L,), jnp.int32),
             pltpu.VMEM((L // GCH, GCH), jnp.int32)]
            + [pltpu.VMEM((GCH, w), jnp.bfloat16)
               for (w, lo, hi) in meta for _ in range(2)]
            + [pltpu.SemaphoreType.DMA]
        ),
        compiler_params=pltpu.CompilerParams(needs_layout_passes=False,
                                             use_tc_tiling_on_sc=False),
    )(sidx_flat, *srcs)


def _scext_body(sidx_flat, chw_hbm, out_hbm, sidx_v, pos_v, b0, b1, sem):
    cid = lax.axis_index("c")
    sid = lax.axis_index("s")
    rpt = BL // NW
    nch = rpt // ECH
    r = sid * NC + cid
    row0 = r * rpt
    b = row0 // L
    lrow0 = row0 - b * L
    cnt = _build_pos(sidx_flat, sidx_v, pos_v, b, BL, lg=ECH_LOG2)
    n_eff = jnp.clip((cnt - lrow0 + ECH - 1) // ECH, 0, nch)
    ch0 = lrow0 // ECH

    @pl.when(n_eff > 0)
    def _():
        def step(i, carry):
            for par, (cur, nxt) in enumerate(((b0, b1), (b1, b0))):
                @pl.when(lax.rem(i, 2) == par)
                def _(cur=cur, nxt=nxt):
                    pltpu.sync_copy(
                        chw_hbm.at[pl.ds(row0 + i * ECH, ECH)], cur)

                    @pl.when(i >= 1)
                    def _():
                        pltpu.make_async_copy(
                            nxt, out_hbm.at[pos_v.at[ch0 + i]], sem).wait()
                    pltpu.async_copy(cur, out_hbm.at[pos_v.at[ch0 + i]],
                                     sem)
            return carry
        lax.fori_loop(0, n_eff, step, 0)
        pltpu.make_async_copy(b0, out_hbm.at[pos_v.at[ch0]], sem).wait()


def _run_scext(sidx_flat, chw2):
    mesh = plsc.VectorSubcoreMesh(core_axis_name="c", subcore_axis_name="s")
    return pl.kernel(
        _scext_body,
        out_type=jax.ShapeDtypeStruct((BL + 16, D), jnp.float32),
        mesh=mesh,
        scratch_types=[
            pltpu.VMEM((L,), jnp.int32),
            pltpu.VMEM((L // ECH, ECH), jnp.int32),
            pltpu.VMEM((ECH, D), jnp.float32),
            pltpu.VMEM((ECH, D), jnp.float32),
            pltpu.SemaphoreType.DMA,
        ],
        compiler_params=pltpu.CompilerParams(needs_layout_passes=False,
                                             use_tc_tiling_on_sc=False),
    )(sidx_flat, chw2)


def _k3_body(czq_ref, czk_ref, cv_ref, bias_ref, whb_ref, counts_ref,
             chw_ref):
    b = pl.program_id(0)
    j = pl.program_id(1)
    cnt = counts_ref[b]

    @pl.when(j * C < cnt)
    def _():
        q = czq_ref[...]
        k = czk_ref[...]
        s = lax.dot_general(q, k, (((1,), (1,)), ((), ())),
                            preferred_element_type=jnp.float32)
        s = s * (1.0 / math.sqrt(Z)) + bias_ref[...]
        col = lax.broadcasted_iota(jnp.int32, (C, C), 1) + j * C
        s = jnp.where(col < cnt, s, NEG)
        m = jnp.max(s, axis=-1, keepdims=True)
        e = jnp.exp(s - m)
        attn = (e / jnp.sum(e, axis=-1, keepdims=True)).astype(jnp.bfloat16)
        h = jnp.dot(attn, cv_ref[...], preferred_element_type=jnp.float32)
        chw_ref[...] = jnp.dot(h.astype(jnp.bfloat16), whb_ref[...],
                               preferred_element_type=jnp.float32)


def _run_k3(czq, czk, cv, bias, whb, counts):
    full = lambda s: pl.BlockSpec(s, lambda b, j: (0,) * len(s))
    return pl.pallas_call(
        _k3_body,
        grid=(B, NCH),
        in_specs=[
            pl.BlockSpec((C, Z), lambda b, j: (b * NCH + j, 0)),
            pl.BlockSpec((C, Z), lambda b, j: (b * NCH + j, 0)),
            pl.BlockSpec((C, D), lambda b, j: (b * NCH + j, 0)),
            full((C, C)), full((D, D)),
            pl.BlockSpec(memory_space=pltpu.SMEM),
        ],
        out_specs=pl.BlockSpec((C, D), lambda b, j: (b * NCH + j, 0)),
        out_shape=jax.ShapeDtypeStruct((BL, D), jnp.float32),
        compiler_params=pltpu.CompilerParams(
            dimension_semantics=("arbitrary", "arbitrary")),
    )(czq, czk, cv, bias, whb, counts)


def _k4_body(x_ref, hxw_ref, g2_ref, out_ref):
    g = g2_ref[0]
    hw = hxw_ref[...]
    out_ref[0] = jnp.where(g > 0.0, x_ref[0] + hw * g, x_ref[0])


def _run_k4(x, hxw, g2):
    R = 512
    NB = L // R
    return pl.pallas_call(
        _k4_body,
        grid=(B, NB),
        in_specs=[
            pl.BlockSpec((1, R, D), lambda b, i: (b, i, 0)),
            pl.BlockSpec((R, D), lambda b, i: (b * NB + i, 0)),
            pl.BlockSpec((1, R, 1), lambda b, i: (b, i, 0)),
        ],
        out_specs=pl.BlockSpec((1, R, D), lambda b, i: (b, i, 0)),
        out_shape=jax.ShapeDtypeStruct((B, L, D), jnp.float32),
        compiler_params=pltpu.CompilerParams(
            dimension_semantics=("arbitrary", "arbitrary")),
    )(x, hxw, g2)


def kernel(x, delta, alpha, beta, gamma, omega, temp, w_conf, w_q, w_k,
           w_v, w_h, rel_pos_bias):
    p = jax.nn.sigmoid(delta[:, :, 0])
    aa = jax.nn.sigmoid(alpha[:, :, 0])
    qd = 1.0 - p * aa
    logq = jnp.log(qd)
    cc = p * beta[:, :, 0] * gamma * math.sqrt(1.0 / N_DIM_CONST)
    t = jnp.arange(C, dtype=jnp.float32)[:, None, None]
    apow = jnp.exp(-t * logq[None])
    ppow = jnp.exp(t * logq[None])
    pq = ppow * qd[None]
    to2d = lambda z: jnp.concatenate([z[:, :, 0], z[:, :, 1]], axis=1)
    a2 = to2d(apow)
    p2 = to2d(ppow)
    pq2 = to2d(pq)
    cb = jnp.concatenate([cc[:, 0], cc[:, 1]])[None, :]
    om = omega[None, :]
    wcs = (w_conf[:, 0] / jnp.exp(temp[0]))[None, :]

    zq2, zk2, v2, g2, sidx = _run_k1(
        x, a2, p2, pq2, cb, om, wcs,
        w_q.astype(jnp.bfloat16), w_k.astype(jnp.bfloat16),
        w_v.astype(jnp.bfloat16))

    sidx_i = sidx[:, :, 0].astype(jnp.int32)
    counts = jnp.max(sidx_i, axis=1)
    sidx_flat = sidx_i.reshape(BL)

    czq2, czk2, cv2 = _run_scgather(
        sidx_flat, (zq2, zk2, v2),
        ((Z, 8, 12), (Z, 12, 16), (D, 0, 8)))

    u3 = jnp.concatenate([rel_pos_bias[::-1], jnp.zeros((1,), jnp.float32)])
    bias = jnp.tile(u3, C)[:C * (2 * C - 1)].reshape(C, 2 * C - 1)
    bias = bias[:, C - 1:]
    chw2 = _run_k3(czq2, czk2, cv2, bias, w_h.astype(jnp.bfloat16), counts)

    hxw = _run_scext(sidx_flat, chw2)

    return _run_k4(x, hxw, g2)


# speedup vs baseline: 1.9970x; 1.1727x over previous
import functools
import math

import jax
import jax.numpy as jnp
from jax import lax
from jax.experimental import pallas as pl
from jax.experimental.pallas import tpu as pltpu
from jax.experimental.pallas import tpu_sc as plsc

D = 1024
Z = 256
C = 128
B = 2
L = 4096
NCH = L // C
BL = B * L
NEG = -50000.0

N_DIM_CONST = 2
_STAGE = 5

NC = 2
NS = 16
NW = NC * NS
GCH = 32
GCH_LOG2 = 5
ECH = 32
ECH_LOG2 = 5


def _tri():
    r = lax.broadcasted_iota(jnp.int32, (C, C), 0)
    c = lax.broadcasted_iota(jnp.int32, (C, C), 1)
    return (c <= r).astype(jnp.float32)


def _k1_body(x_ref, a_ref, p_ref, pq_ref, cb_ref, om_ref, wcs_ref,
             wq_ref, wk_ref, wv_ref,
             zq_ref, zk_ref, v_ref, g2_ref, sidx_ref, carry_ref, cnt_ref):
    j = pl.program_id(1)

    @pl.when(j == 0)
    def _():
        carry_ref[...] = jnp.zeros_like(carry_ref)
        cnt_ref[0] = 0.0

    x = x_ref[0]
    x2 = jnp.concatenate([x, x], axis=1)
    tri = _tri()
    y = x2 * a_ref[...]
    cum = jnp.dot(tri, y, preferred_element_type=jnp.float32)
    s = cum * p_ref[...] + carry_ref[0:1, :] * pq_ref[...]
    carry_ref[0:1, :] = s[C - 1:C, :]
    sc = s * cb_ref[...]
    conv = sc[:, :D] + sc[:, D:]
    mx = conv + x * om_ref[...]
    mx = mx * jax.nn.sigmoid(mx)

    logit = jnp.sum(mx * wcs_ref[...], axis=1, keepdims=True)
    sel = (logit > 0.0).astype(jnp.float32)
    g2_ref[0] = jax.nn.sigmoid(logit) * sel
    csel = jnp.dot(tri, sel, preferred_element_type=jnp.float32)
    sidx_ref[0] = (csel + cnt_ref[0]) * sel
    cnt_ref[0] = cnt_ref[0] + csel[C - 1, 0]

    mxb = mx.astype(jnp.bfloat16)
    zq_ref[...] = jnp.dot(mxb, wq_ref[...],
                          preferred_element_type=jnp.float32
                          ).astype(jnp.bfloat16)
    zk_ref[...] = jnp.dot(mxb, wk_ref[...],
                          preferred_element_type=jnp.float32
                          ).astype(jnp.bfloat16)
    vv = jnp.dot(x.astype(jnp.bfloat16), wv_ref[...],
                 preferred_element_type=jnp.float32)
    v_ref[...] = vv * jax.nn.sigmoid(vv)


def _run_k1(x, a, p, pq, cb, om, wcs, wqb, wkb, wvb):
    full = lambda s: pl.BlockSpec(s, lambda b, j: (0,) * len(s))
    return pl.pallas_call(
        _k1_body,
        grid=(B, NCH),
        in_specs=[
            pl.BlockSpec((1, C, D), lambda b, j: (b, j, 0)),
            full((C, 2 * D)), full((C, 2 * D)), full((C, 2 * D)),
            full((1, 2 * D)), full((1, D)), full((1, D)),
            full((D, Z)), full((D, Z)), full((D, D)),
        ],
        out_specs=[
            pl.BlockSpec((C, Z), lambda b, j: (b * NCH + j, 0)),
            pl.BlockSpec((C, Z), lambda b, j: (b * NCH + j, 0)),
            pl.BlockSpec((C, D), lambda b, j: (b * NCH + j, 0)),
            pl.BlockSpec((1, C, 1), lambda b, j: (b, j, 0)),
            pl.BlockSpec((1, C, 1), lambda b, j: (b, j, 0)),
        ],
        out_shape=[
            jax.ShapeDtypeStruct((BL, Z), jnp.bfloat16),
            jax.ShapeDtypeStruct((BL, Z), jnp.bfloat16),
            jax.ShapeDtypeStruct((BL, D), jnp.float32),
            jax.ShapeDtypeStruct((B, L, 1), jnp.float32),
            jax.ShapeDtypeStruct((B, L, 1), jnp.float32),
        ],
        scratch_shapes=[
            pltpu.VMEM((8, 2 * D), jnp.float32),
            pltpu.SMEM((1,), jnp.float32),
        ],
        compiler_params=pltpu.CompilerParams(
            dimension_semantics=("arbitrary", "arbitrary")),
    )(x, a, p, pq, cb, om, wcs, wqb, wkb, wvb)


def _build_pos(sidx_flat, sidx_v, pos_v, b, init_val, lg=GCH_LOG2):
    w = 1 << lg
    pltpu.sync_copy(sidx_flat.at[pl.ds(b * L, L)], sidx_v)

    def zf(i, c):
        for k in range(w // 16):
            pos_v[i, pl.ds(k * 16, 16)] = jnp.full((16,), init_val,
                                                   jnp.int32)
        return c
    lax.fori_loop(0, L // w, zf, 0)

    base = b * L

    def scat(i, mvec):
        ranks = sidx_v[pl.ds(i * 16, 16)]
        r0 = jnp.maximum(ranks - 1, 0)
        tvec = lax.iota(jnp.int32, 16) + (i * 16 + base)
        plsc.store_scatter(pos_v,
                           [lax.shift_right_logical(r0, lg),
                            lax.bitwise_and(r0, (1 << lg) - 1)],
                           tvec, mask=ranks > 0)
        return jnp.maximum(mvec, ranks)
    mvec = lax.fori_loop(0, L // 16, scat, jnp.zeros((16,), jnp.int32))
    return jnp.max(mvec)


def _scgather_body(meta, sidx_flat, *rest):
    nsrc = len(meta)
    srcs = rest[:nsrc]
    outs = rest[nsrc:2 * nsrc]
    sidx_v = rest[2 * nsrc]
    pos_v = rest[2 * nsrc + 1]
    bufs = rest[2 * nsrc + 2:2 * nsrc + 2 + 2 * nsrc]
    sem = rest[-1]
    cid = lax.axis_index("c")
    sid = lax.axis_index("s")
    for t, (w, lo, hi, dt) in enumerate(meta):
        ntiles = (hi - lo) * NC
        rpt = BL // ntiles
        nch = rpt // GCH
        src, out, b0, b1 = srcs[t], outs[t], bufs[2 * t], bufs[2 * t + 1]

        @pl.when((sid >= lo) & (sid < hi))
        def _(src=src, out=out, b0=b0, b1=b1, lo=lo, rpt=rpt, nch=nch):
            r = (sid - lo) * NC + cid
            row0 = r * rpt
            b = row0 // L
            lrow0 = row0 - b * L
            cnt = _build_pos(sidx_flat, sidx_v, pos_v, b, 0)
            bound = ((cnt + C - 1) // C) * C
            n_eff = jnp.clip((bound - lrow0 + GCH - 1) // GCH, 0, nch)
            ch0 = lrow0 // GCH

            @pl.when(n_eff > 0)
            def _():
                pltpu.async_copy(src.at[pos_v.at[ch0]], b0, sem)

                def step(i, carry):
                    for par, (cur, nxt) in enumerate(((b0, b1), (b1, b0))):
                        @pl.when(lax.rem(i, 2) == par)
                        def _(cur=cur, nxt=nxt):
                            @pl.when(i + 1 < n_eff)
                            def _():
                                pltpu.async_copy(
                                    src.at[pos_v.at[ch0 + i + 1]], nxt, sem)
                            pltpu.make_async_copy(
                                src.at[pos_v.at[ch0 + i]], cur, sem).wait()
                            pltpu.sync_copy(
                                cur, out.at[pl.ds(row0 + i * GCH, GCH)])
                    return carry
                lax.fori_loop(0, n_eff, step, 0)


def _run_scgather(sidx_flat, srcs, meta):
    mesh = plsc.VectorSubcoreMesh(core_axis_name="c", subcore_axis_name="s")
    return pl.kernel(
        functools.partial(_scgather_body, meta),
        out_type=tuple(jax.ShapeDtypeStruct((BL, w), dt)
                       for (w, lo, hi, dt) in meta),
        mesh=mesh,
        scratch_types=(
            [pltpu.VMEM((L,), jnp.int32),
             pltpu.VMEM((L // GCH, GCH), jnp.int32)]
            + [pltpu.VMEM((GCH, w), dt)
               for (w, lo, hi, dt) in meta for _ in range(2)]
            + [pltpu.SemaphoreType.DMA]
        ),
        compiler_params=pltpu.CompilerParams(needs_layout_passes=False,
                                             use_tc_tiling_on_sc=False),
    )(sidx_flat, *srcs)


def _scext_body(sidx_flat, chw_hbm, out_hbm, sidx_v, pos_v, b0, b1, sem):
    cid = lax.axis_index("c")
    sid = lax.axis_index("s")
    rpt = BL // NW
    nch = rpt // ECH
    r = sid * NC + cid
    row0 = r * rpt
    b = row0 // L
    lrow0 = row0 - b * L
    cnt = _build_pos(sidx_flat, sidx_v, pos_v, b, BL, lg=ECH_LOG2)
    n_eff = jnp.clip((cnt - lrow0 + ECH - 1) // ECH, 0, nch)
    ch0 = lrow0 // ECH

    @pl.when(n_eff > 0)
    def _():
        def step(i, carry):
            for par, (cur, nxt) in enumerate(((b0, b1), (b1, b0))):
                @pl.when(lax.rem(i, 2) == par)
                def _(cur=cur, nxt=nxt):
                    pltpu.sync_copy(
                        chw_hbm.at[pl.ds(row0 + i * ECH, ECH)], cur)

                    @pl.when(i >= 1)
                    def _():
                        pltpu.make_async_copy(
                            nxt, out_hbm.at[pos_v.at[ch0 + i]], sem).wait()
                    pltpu.async_copy(cur, out_hbm.at[pos_v.at[ch0 + i]],
                                     sem)
            return carry
        lax.fori_loop(0, n_eff, step, 0)
        pltpu.make_async_copy(b0, out_hbm.at[pos_v.at[ch0]], sem).wait()


def _run_scext(sidx_flat, chw2):
    mesh = plsc.VectorSubcoreMesh(core_axis_name="c", subcore_axis_name="s")
    return pl.kernel(
        _scext_body,
        out_type=jax.ShapeDtypeStruct((BL + 16, D), jnp.float32),
        mesh=mesh,
        scratch_types=[
            pltpu.VMEM((L,), jnp.int32),
            pltpu.VMEM((L // ECH, ECH), jnp.int32),
            pltpu.VMEM((ECH, D), jnp.float32),
            pltpu.VMEM((ECH, D), jnp.float32),
            pltpu.SemaphoreType.DMA,
        ],
        compiler_params=pltpu.CompilerParams(needs_layout_passes=False,
                                             use_tc_tiling_on_sc=False),
    )(sidx_flat, chw2)


def _k3_body(czq_ref, czk_ref, cv_ref, bias_ref, whb_ref, counts_ref,
             chw_ref):
    b = pl.program_id(0)
    j = pl.program_id(1)
    cnt = counts_ref[b]

    @pl.when(j * C < cnt)
    def _():
        q = czq_ref[...]
        k = czk_ref[...]
        s = lax.dot_general(q, k, (((1,), (1,)), ((), ())),
                            preferred_element_type=jnp.float32)
        s = s * (1.0 / math.sqrt(Z)) + bias_ref[...]
        col = lax.broadcasted_iota(jnp.int32, (C, C), 1) + j * C
        s = jnp.where(col < cnt, s, NEG)
        m = jnp.max(s, axis=-1, keepdims=True)
        e = jnp.exp(s - m)
        attn = (e / jnp.sum(e, axis=-1, keepdims=True)).astype(jnp.bfloat16)
        h = jnp.dot(attn, cv_ref[...].astype(jnp.bfloat16),
                    preferred_element_type=jnp.float32)
        chw_ref[...] = jnp.dot(h.astype(jnp.bfloat16), whb_ref[...],
                               preferred_element_type=jnp.float32)


def _run_k3(czq, czk, cv, bias, whb, counts):
    full = lambda s: pl.BlockSpec(s, lambda b, j: (0,) * len(s))
    return pl.pallas_call(
        _k3_body,
        grid=(B, NCH),
        in_specs=[
            pl.BlockSpec((C, Z), lambda b, j: (b * NCH + j, 0)),
            pl.BlockSpec((C, Z), lambda b, j: (b * NCH + j, 0)),
            pl.BlockSpec((C, D), lambda b, j: (b * NCH + j, 0)),
            full((C, C)), full((D, D)),
            pl.BlockSpec(memory_space=pltpu.SMEM),
        ],
        out_specs=pl.BlockSpec((C, D), lambda b, j: (b * NCH + j, 0)),
        out_shape=jax.ShapeDtypeStruct((BL, D), jnp.float32),
        compiler_params=pltpu.CompilerParams(
            dimension_semantics=("arbitrary", "arbitrary")),
    )(czq, czk, cv, bias, whb, counts)


def _k4_body(x_ref, hxw_ref, g2_ref, out_ref):
    g = g2_ref[0]
    hw = hxw_ref[...]
    out_ref[0] = jnp.where(g > 0.0, x_ref[0] + hw * g, x_ref[0])


def _run_k4(x, hxw, g2):
    R = 512
    NB = L // R
    return pl.pallas_call(
        _k4_body,
        grid=(B, NB),
        in_specs=[
            pl.BlockSpec((1, R, D), lambda b, i: (b, i, 0)),
            pl.BlockSpec((R, D), lambda b, i: (b * NB + i, 0)),
            pl.BlockSpec((1, R, 1), lambda b, i: (b, i, 0)),
        ],
        out_specs=pl.BlockSpec((1, R, D), lambda b, i: (b, i, 0)),
        out_shape=jax.ShapeDtypeStruct((B, L, D), jnp.float32),
        compiler_params=pltpu.CompilerParams(
            dimension_semantics=("arbitrary", "arbitrary")),
    )(x, hxw, g2)


def kernel(x, delta, alpha, beta, gamma, omega, temp, w_conf, w_q, w_k,
           w_v, w_h, rel_pos_bias):
    p = jax.nn.sigmoid(delta[:, :, 0])
    aa = jax.nn.sigmoid(alpha[:, :, 0])
    qd = 1.0 - p * aa
    logq = jnp.log(qd)
    cc = p * beta[:, :, 0] * gamma * math.sqrt(1.0 / N_DIM_CONST)
    t = jnp.arange(C, dtype=jnp.float32)[:, None, None]
    apow = jnp.exp(-t * logq[None])
    ppow = jnp.exp(t * logq[None])
    pq = ppow * qd[None]
    to2d = lambda z: jnp.concatenate([z[:, :, 0], z[:, :, 1]], axis=1)
    a2 = to2d(apow)
    p2 = to2d(ppow)
    pq2 = to2d(pq)
    cb = jnp.concatenate([cc[:, 0], cc[:, 1]])[None, :]
    om = omega[None, :]
    wcs = (w_conf[:, 0] / jnp.exp(temp[0]))[None, :]

    zq2, zk2, v2, g2, sidx = _run_k1(
        x, a2, p2, pq2, cb, om, wcs,
        w_q.astype(jnp.bfloat16), w_k.astype(jnp.bfloat16),
        w_v.astype(jnp.bfloat16))

    sidx_i = sidx[:, :, 0].astype(jnp.int32)
    counts = jnp.max(sidx_i, axis=1)
    sidx_flat = sidx_i.reshape(BL)

    czq2, czk2, cv2 = _run_scgather(
        sidx_flat, (zq2, zk2, v2),
        ((Z, 8, 12, jnp.bfloat16), (Z, 12, 16, jnp.bfloat16),
         (D, 0, 8, jnp.float32)))

    u3 = jnp.concatenate([rel_pos_bias[::-1], jnp.zeros((1,), jnp.float32)])
    bias = jnp.tile(u3, C)[:C * (2 * C - 1)].reshape(C, 2 * C - 1)
    bias = bias[:, C - 1:]
    chw2 = _run_k3(czq2, czk2, cv2, bias, w_h.astype(jnp.bfloat16), counts)

    hxw = _run_scext(sidx_flat, chw2)

    return _run_k4(x, hxw, g2)
